# baseline (speedup 1.0000x reference)
import functools

import jax
import jax.numpy as jnp
from jax import lax
from jax.experimental import pallas as pl
from jax.experimental.pallas import tpu as pltpu
from jax.experimental.pallas import tpu_sc as plsc

B, S, D = 4096, 200, 64
NC, NS = 2, 16
NW = NC * NS
R = S * D
DCH = 2
SPW = R // NW // DCH
RPW = SPW * DCH
NBUF = 4
G = SPW // NBUF
LANES = 16
NSL = B // LANES

_mesh = plsc.VectorSubcoreMesh(
    core_axis_name="c", subcore_axis_name="s", num_cores=NC, num_subcores=NS
)


@functools.partial(
    pl.kernel,
    out_type=jax.ShapeDtypeStruct((R, B), jnp.float32),
    mesh=_mesh,
    compiler_params=pltpu.CompilerParams(use_tc_tiling_on_sc=True),
    scratch_types=(
        [pltpu.VMEM((RPW, LANES), jnp.float32)]
        + [pltpu.VMEM((DCH, B), jnp.float32) for _ in range(2 * NBUF)]
        + [pltpu.SemaphoreType.DMA for _ in range(2 * NBUF)]
    ),
)
def _pos_add_t(xt_hbm, ex_hbm, out_hbm, emb_v, *rest):
    in_bufs = rest[:NBUF]
    out_bufs = rest[NBUF : 2 * NBUF]
    in_sems = rest[2 * NBUF : 3 * NBUF]
    out_sems = rest[3 * NBUF :]

    wid = lax.axis_index("s") * NC + lax.axis_index("c")
    base = wid * SPW

    pltpu.sync_copy(ex_hbm.at[pl.ds(base * DCH, RPW)], emb_v)

    def start_in(k, i):
        r0 = (base + i) * DCH
        pltpu.async_copy(xt_hbm.at[pl.ds(r0, DCH)], in_bufs[k], in_sems[k])

    def wait_in(k):
        pltpu.make_async_copy(
            xt_hbm.at[pl.ds(0, DCH)], in_bufs[k], in_sems[k]
        ).wait()

    def start_out(k, i):
        r0 = (base + i) * DCH
        pltpu.async_copy(out_bufs[k], out_hbm.at[pl.ds(r0, DCH)], out_sems[k])

    def wait_out(k):
        pltpu.make_async_copy(
            out_bufs[k], out_hbm.at[pl.ds(0, DCH)], out_sems[k]
        ).wait()

    def add_slab(k, i):
        src = in_bufs[k]
        dst = out_bufs[k]
        vecs = [emb_v[i * DCH + j] for j in range(DCH)]

        def body(u, _):
            sl = pl.ds(u * LANES, LANES)
            for j in range(DCH):
                dst[j, sl] = src[j, sl] + vecs[j]
            return 0

        lax.fori_loop(0, NSL, body, 0)

    for k in range(NBUF):
        start_in(k, k)

    for k in range(NBUF):
        wait_in(k)
        add_slab(k, k)
        start_out(k, k)
        start_in(k, NBUF + k)

    def group(g, _):
        for k in range(NBUF):
            i = g * NBUF + k
            wait_in(k)
            wait_out(k)
            add_slab(k, i)
            start_out(k, i)
            start_in(k, i + NBUF)
        return 0

    lax.fori_loop(1, G - 1, group, 0)

    for k in range(NBUF):
        i = (G - 1) * NBUF + k
        wait_in(k)
        wait_out(k)
        add_slab(k, i)
        start_out(k, i)
    for k in range(NBUF):
        wait_out(k)


def kernel(x, emb):
    xt = jnp.reshape(jnp.transpose(x, (1, 2, 0)), (R, B))
    ex = jnp.broadcast_to(jnp.reshape(emb, (R, 1)), (R, LANES))
    out_t = _pos_add_t(xt, ex)
    return jnp.transpose(jnp.reshape(out_t, (S, D, B)), (2, 0, 1))


# speedup vs baseline: 1.1756x; 1.1756x over previous
import functools

import jax
import jax.numpy as jnp
from jax import lax
from jax.experimental import pallas as pl
from jax.experimental.pallas import tpu as pltpu
from jax.experimental.pallas import tpu_sc as plsc

B, S, D = 4096, 200, 64
NC, NS = 2, 16
NW = NC * NS
R = S * D
DCH = 4
SPS = D // DCH
SPW = (S * SPS) // NW
RPW = SPW * DCH
NBUF = 2
G = SPW // NBUF
LANES = 16
NSL = B // LANES

_mesh = plsc.VectorSubcoreMesh(
    core_axis_name="c", subcore_axis_name="s", num_cores=NC, num_subcores=NS
)


@functools.partial(
    pl.kernel,
    out_type=jax.ShapeDtypeStruct((S, D, B), jnp.float32),
    mesh=_mesh,
    compiler_params=pltpu.CompilerParams(use_tc_tiling_on_sc=True),
    scratch_types=(
        [pltpu.VMEM((RPW, LANES), jnp.float32)]
        + [pltpu.VMEM((DCH, B), jnp.float32) for _ in range(2 * NBUF)]
        + [pltpu.SemaphoreType.DMA for _ in range(2 * NBUF)]
    ),
)
def _pos_add_t(xt_hbm, ex_hbm, out_hbm, emb_v, *rest):
    in_bufs = rest[:NBUF]
    out_bufs = rest[NBUF : 2 * NBUF]
    in_sems = rest[2 * NBUF : 3 * NBUF]
    out_sems = rest[3 * NBUF :]

    wid = lax.axis_index("s") * NC + lax.axis_index("c")
    base = wid * SPW

    pltpu.sync_copy(ex_hbm.at[pl.ds(base * DCH, RPW)], emb_v)

    def coords(i):
        slab = base + i
        return slab // SPS, (slab % SPS) * DCH

    def start_in(k, i):
        s, d0 = coords(i)
        pltpu.async_copy(xt_hbm.at[s, pl.ds(d0, DCH)], in_bufs[k], in_sems[k])

    def wait_in(k):
        pltpu.make_async_copy(
            xt_hbm.at[0, pl.ds(0, DCH)], in_bufs[k], in_sems[k]
        ).wait()

    def start_out(k, i):
        s, d0 = coords(i)
        pltpu.async_copy(out_bufs[k], out_hbm.at[s, pl.ds(d0, DCH)], out_sems[k])

    def wait_out(k):
        pltpu.make_async_copy(
            out_bufs[k], out_hbm.at[0, pl.ds(0, DCH)], out_sems[k]
        ).wait()

    def add_slab(k, i):
        src = in_bufs[k]
        dst = out_bufs[k]
        vecs = [emb_v[i * DCH + j] for j in range(DCH)]

        @plsc.parallel_loop(0, NSL, unroll=8)
        def body(u):
            sl = pl.ds(u * LANES, LANES)
            for j in range(DCH):
                dst[j, sl] = src[j, sl] + vecs[j]

    for k in range(NBUF):
        start_in(k, k)

    for k in range(NBUF):
        wait_in(k)
        add_slab(k, k)
        start_out(k, k)
        start_in(k, NBUF + k)

    def group(g, _):
        for k in range(NBUF):
            i = g * NBUF + k
            wait_in(k)
            wait_out(k)
            add_slab(k, i)
            start_out(k, i)
            start_in(k, i + NBUF)
        return 0

    lax.fori_loop(1, G - 1, group, 0)

    for k in range(NBUF):
        i = (G - 1) * NBUF + k
        wait_in(k)
        wait_out(k)
        add_slab(k, i)
        start_out(k, i)
    for k in range(NBUF):
        wait_out(k)


def kernel(x, emb):
    xt = jnp.transpose(x, (1, 2, 0))
    ex = jnp.broadcast_to(jnp.reshape(emb, (R, 1)), (R, LANES))
    out_t = _pos_add_t(xt, ex)
    return jnp.transpose(out_t, (2, 0, 1))


# speedup vs baseline: 1.1805x; 1.0042x over previous
import functools

import jax
import jax.numpy as jnp
from jax import lax
from jax.experimental import pallas as pl
from jax.experimental.pallas import tpu as pltpu
from jax.experimental.pallas import tpu_sc as plsc

B, S, D = 4096, 200, 64
NC, NS = 2, 16
NW = NC * NS
R = S * D
DCH = 4
SPS = D // DCH
SPW = (S * SPS) // NW
RPW = SPW * DCH
NBUF = 3
G = SPW // NBUF
TAIL = SPW - G * NBUF
LANES = 16
NSL = B // LANES

_mesh = plsc.VectorSubcoreMesh(
    core_axis_name="c", subcore_axis_name="s", num_cores=NC, num_subcores=NS
)


@functools.partial(
    pl.kernel,
    out_type=jax.ShapeDtypeStruct((S, D, B), jnp.float32),
    mesh=_mesh,
    compiler_params=pltpu.CompilerParams(use_tc_tiling_on_sc=True),
    scratch_types=(
        [pltpu.VMEM((DCH, B), jnp.float32) for _ in range(2 * NBUF)]
        + [pltpu.VMEM((DCH, LANES), jnp.float32) for _ in range(NBUF)]
        + [pltpu.SemaphoreType.DMA for _ in range(3 * NBUF)]
    ),
)
def _pos_add_t(xt_hbm, ex_hbm, out_hbm, *rest):
    in_bufs = rest[:NBUF]
    out_bufs = rest[NBUF : 2 * NBUF]
    emb_bufs = rest[2 * NBUF : 3 * NBUF]
    in_sems = rest[3 * NBUF : 4 * NBUF]
    out_sems = rest[4 * NBUF : 5 * NBUF]
    emb_sems = rest[5 * NBUF :]

    wid = lax.axis_index("s") * NC + lax.axis_index("c")
    base = wid * SPW

    def coords(i):
        slab = base + i
        return slab // SPS, (slab % SPS) * DCH

    def start_in(k, i):
        s, d0 = coords(i)
        pltpu.async_copy(xt_hbm.at[s, pl.ds(d0, DCH)], in_bufs[k], in_sems[k])
        pltpu.async_copy(
            ex_hbm.at[pl.ds((base + i) * DCH, DCH)], emb_bufs[k], emb_sems[k]
        )

    def wait_in(k):
        pltpu.make_async_copy(
            xt_hbm.at[0, pl.ds(0, DCH)], in_bufs[k], in_sems[k]
        ).wait()
        pltpu.make_async_copy(
            ex_hbm.at[pl.ds(0, DCH)], emb_bufs[k], emb_sems[k]
        ).wait()

    def start_out(k, i):
        s, d0 = coords(i)
        pltpu.async_copy(out_bufs[k], out_hbm.at[s, pl.ds(d0, DCH)], out_sems[k])

    def wait_out(k):
        pltpu.make_async_copy(
            out_bufs[k], out_hbm.at[0, pl.ds(0, DCH)], out_sems[k]
        ).wait()

    def add_slab(k, i):
        src = in_bufs[k]
        dst = out_bufs[k]
        vecs = [emb_bufs[k][j] for j in range(DCH)]

        @plsc.parallel_loop(0, NSL, unroll=8)
        def body(u):
            sl = pl.ds(u * LANES, LANES)
            for j in range(DCH):
                dst[j, sl] = src[j, sl] + vecs[j]

    for k in range(NBUF):
        start_in(k, k)

    for k in range(NBUF):
        wait_in(k)
        add_slab(k, k)
        start_out(k, k)
        start_in(k, NBUF + k)

    def group(g, _):
        for k in range(NBUF):
            i = g * NBUF + k
            wait_in(k)
            wait_out(k)
            add_slab(k, i)
            start_out(k, i)
            start_in(k, i + NBUF)
        return 0

    lax.fori_loop(1, G - 1, group, 0)

    for k in range(NBUF):
        i = (G - 1) * NBUF + k
        wait_in(k)
        wait_out(k)
        add_slab(k, i)
        start_out(k, i)
        if i + NBUF < SPW:
            start_in(k, i + NBUF)

    for i in range(G * NBUF, SPW):
        k = i % NBUF
        wait_in(k)
        wait_out(k)
        add_slab(k, i)
        start_out(k, i)
    for k in range(NBUF):
        wait_out(k)


def kernel(x, emb):
    xt = jnp.transpose(x, (1, 2, 0))
    ex = jnp.broadcast_to(jnp.reshape(emb, (R, 1)), (R, LANES))
    out_t = _pos_add_t(xt, ex)
    return jnp.transpose(out_t, (2, 0, 1))
